# Initial kernel scaffold; baseline (speedup 1.0000x reference)
import jax
import jax.numpy as jnp
from jax.experimental import pallas as pl


def kernel(x, edge_index, W_self0, W_neigh0, b0, W_self1, W_neigh1, b1):
    raise NotImplementedError("write your pallas kernel here")


# speedup vs baseline: 2.2068x; 2.2068x over previous
import functools

import jax
import jax.numpy as jnp
from jax import lax
from jax.experimental import pallas as pl
from jax.experimental.pallas import tpu as pltpu
from jax.experimental.pallas import tpu_sc as plsc

N = 10000
E = 160000
D = 256
H = 128
NC = 2
NS = 16
K = 64
NCH = 160
EPT = K * NCH
E_PAD = EPT * NS
STRIPE = 632
N_PAD = STRIPE * NS

BN = 1000


def _mesh():
    return plsc.VectorSubcoreMesh(
        core_axis_name="c", subcore_axis_name="s",
        num_cores=NC, num_subcores=NS)


def _sc_body(src2, dst, table, z128, s_out, acc, sb, db, rows, gsem):
    c = lax.axis_index("c")
    s = lax.axis_index("s")
    sbase = c * E_PAD + s * EPT
    dbase = s * EPT

    pltpu.sync_copy(z128, acc.at[pl.ds(s * STRIPE, STRIPE)])
    plsc.subcore_barrier()

    @pl.loop(0, NCH)
    def _(j):
        pltpu.sync_copy(src2.at[pl.ds(sbase + j * K, K)], sb)
        pltpu.sync_copy(dst.at[pl.ds(dbase + j * K, K)], db)
        pltpu.async_copy(table.at[sb], rows, gsem).wait()
        pltpu.sync_copy(rows, acc.at[db], add=True)

    plsc.subcore_barrier()

    sl = pl.ds(s * STRIPE, STRIPE)
    osl = pl.ds(c * N_PAD + s * STRIPE, STRIPE)
    pltpu.sync_copy(acc.at[sl], s_out.at[osl])


def _make_sc_call():
    return pl.kernel(
        _sc_body,
        out_type=jax.ShapeDtypeStruct((NC * N_PAD, H), jnp.float32),
        mesh=_mesh(),
        scratch_types=[
            pltpu.VMEM_SHARED((N_PAD, H), jnp.float32),
            pltpu.VMEM((K,), jnp.int32),
            pltpu.VMEM((K,), jnp.int32),
            pltpu.VMEM((K, H), jnp.float32),
            pltpu.SemaphoreType.DMA,
        ],
        name="sage_sc_scatter",
    )


EPT_D = E_PAD // (NC * NS)
NCH_D = EPT_D // K


def _sc_deg_body(dst, ones128, z128, deg_out, acc, db, ones_v, gsem):
    del gsem
    c = lax.axis_index("c")
    s = lax.axis_index("s")
    dbase = (c * NS + s) * EPT_D

    pltpu.sync_copy(z128, acc.at[pl.ds(s * STRIPE, STRIPE)])
    pltpu.sync_copy(ones128, ones_v)
    plsc.subcore_barrier()

    @pl.loop(0, NCH_D)
    def _(j):
        pltpu.sync_copy(dst.at[pl.ds(dbase + j * K, K)], db)
        pltpu.sync_copy(ones_v, acc.at[db], add=True)

    plsc.subcore_barrier()

    sl = pl.ds(s * STRIPE, STRIPE)
    osl = pl.ds(c * N_PAD + s * STRIPE, STRIPE)
    pltpu.sync_copy(acc.at[sl], deg_out.at[osl])


def _make_sc_deg_call():
    return pl.kernel(
        _sc_deg_body,
        out_type=jax.ShapeDtypeStruct((NC * N_PAD, H), jnp.float32),
        mesh=_mesh(),
        scratch_types=[
            pltpu.VMEM_SHARED((N_PAD, H), jnp.float32),
            pltpu.VMEM((K,), jnp.int32),
            pltpu.VMEM((K, H), jnp.float32),
            pltpu.SemaphoreType.DMA,
        ],
        name="sage_sc_degree",
    )


def _tc_in_body(x_ref, wn_ref, ws_ref, b_ref, g_ref, p_ref):
    xb = x_ref[...]
    wn = wn_ref[...]
    g_ref[0] = jnp.dot(xb, wn[:, :H], preferred_element_type=jnp.float32)
    g_ref[1] = jnp.dot(xb, wn[:, H:], preferred_element_type=jnp.float32)
    p_ref[...] = (jnp.dot(xb, ws_ref[...], preferred_element_type=jnp.float32)
                  + b_ref[...])


def _tc_in(x, wn, ws, b):
    return pl.pallas_call(
        _tc_in_body,
        grid=(N // BN,),
        in_specs=[
            pl.BlockSpec((BN, D), lambda i: (i, 0)),
            pl.BlockSpec((D, D), lambda i: (0, 0)),
            pl.BlockSpec((D, D), lambda i: (0, 0)),
            pl.BlockSpec((1, D), lambda i: (0, 0)),
        ],
        out_specs=[
            pl.BlockSpec((NC, BN, H), lambda i: (0, i, 0)),
            pl.BlockSpec((BN, D), lambda i: (i, 0)),
        ],
        out_shape=[
            jax.ShapeDtypeStruct((NC, N, H), jnp.float32),
            jax.ShapeDtypeStruct((N, D), jnp.float32),
        ],
        name="sage_tc_in",
    )(x, wn, ws, b)


def _tc_mid_body(p_ref, s_ref, deg_ref, wn_ref, ws_ref, b_ref, g_ref, o_ref):
    deg = deg_ref[0, :, 0:1] + deg_ref[1, :, 0:1]
    rd = 1.0 / jnp.maximum(deg, 1.0)
    sc = jnp.concatenate([s_ref[0], s_ref[1]], axis=1)
    h1 = jnp.maximum(p_ref[...] + sc * rd, 0.0)
    wn = wn_ref[...]
    g_ref[0] = jnp.dot(h1, wn[:, :H], preferred_element_type=jnp.float32)
    g_ref[1] = jnp.dot(h1, wn[:, H:], preferred_element_type=jnp.float32)
    o_ref[...] = (jnp.dot(h1, ws_ref[...], preferred_element_type=jnp.float32)
                  + b_ref[...])


def _tc_mid(p0, s0, deg16, wn, ws, b):
    return pl.pallas_call(
        _tc_mid_body,
        grid=(N // BN,),
        in_specs=[
            pl.BlockSpec((BN, D), lambda i: (i, 0)),
            pl.BlockSpec((NC, BN, H), lambda i: (0, i, 0)),
            pl.BlockSpec((NC, BN, H), lambda i: (0, i, 0)),
            pl.BlockSpec((D, D), lambda i: (0, 0)),
            pl.BlockSpec((D, D), lambda i: (0, 0)),
            pl.BlockSpec((1, D), lambda i: (0, 0)),
        ],
        out_specs=[
            pl.BlockSpec((NC, BN, H), lambda i: (0, i, 0)),
            pl.BlockSpec((BN, D), lambda i: (i, 0)),
        ],
        out_shape=[
            jax.ShapeDtypeStruct((NC, N, H), jnp.float32),
            jax.ShapeDtypeStruct((N, D), jnp.float32),
        ],
        name="sage_tc_mid",
    )(p0, s0, deg16, wn, ws, b)


def _tc_out_body(p_ref, s_ref, deg_ref, o_ref):
    deg = deg_ref[0, :, 0:1] + deg_ref[1, :, 0:1]
    rd = 1.0 / jnp.maximum(deg, 1.0)
    sc = jnp.concatenate([s_ref[0], s_ref[1]], axis=1)
    o_ref[...] = p_ref[...] + sc * rd


def _tc_out(p1, s1, deg16):
    return pl.pallas_call(
        _tc_out_body,
        grid=(N // BN,),
        in_specs=[
            pl.BlockSpec((BN, D), lambda i: (i, 0)),
            pl.BlockSpec((NC, BN, H), lambda i: (0, i, 0)),
            pl.BlockSpec((NC, BN, H), lambda i: (0, i, 0)),
        ],
        out_specs=pl.BlockSpec((BN, D), lambda i: (i, 0)),
        out_shape=jax.ShapeDtypeStruct((N, D), jnp.float32),
        name="sage_tc_out",
    )(p1, s1, deg16)


def kernel(x, edge_index, W_self0, W_neigh0, b0, W_self1, W_neigh1, b1):
    src = edge_index[0]
    dst = edge_index[1]
    pad = E_PAD - E
    src_p = jnp.concatenate([src, jnp.zeros((pad,), jnp.int32)])
    dst_p = jnp.concatenate([dst, jnp.full((pad,), N, jnp.int32)])
    src2 = jnp.concatenate([src_p, src_p + N])

    z128 = jnp.zeros((STRIPE, H), jnp.float32)
    ones128 = jnp.ones((K, H), jnp.float32)

    b0r = b0.reshape(1, D)
    b1r = b1.reshape(1, D)

    sc_scatter = _make_sc_call()
    sc_degree = _make_sc_deg_call()

    deg = sc_degree(dst_p, ones128, z128).reshape(NC, N_PAD, H)
    g0, p0 = _tc_in(x, W_neigh0, W_self0, b0r)
    s0 = sc_scatter(src2, dst_p, g0.reshape(NC * N, H), z128)
    s0 = s0.reshape(NC, N_PAD, H)
    g1, p1 = _tc_mid(p0, s0, deg, W_neigh1, W_self1, b1r)
    s1 = sc_scatter(src2, dst_p, g1.reshape(NC * N, H), z128)
    s1 = s1.reshape(NC, N_PAD, H)
    return _tc_out(p1, s1, deg)


# speedup vs baseline: 3.4526x; 1.5645x over previous
import functools

import jax
import jax.numpy as jnp
from jax import lax
from jax.experimental import pallas as pl
from jax.experimental.pallas import tpu as pltpu
from jax.experimental.pallas import tpu_sc as plsc

N = 10000
E = 160000
D = 256
H = 128
NC = 2
NS = 16
K = 32
NSLOT = 4
NCH = 320
EPT = K * NCH
E_PAD = EPT * NS
STRIPE = 632
N_PAD = STRIPE * NS

BN = 1000


def _mesh():
    return plsc.VectorSubcoreMesh(
        core_axis_name="c", subcore_axis_name="s",
        num_cores=NC, num_subcores=NS)


def _sc_body(src2, dst, table, z128, s_out, acc, *refs):
    slots = [refs[i * 6:(i + 1) * 6] for i in range(NSLOT)]

    c = lax.axis_index("c")
    s = lax.axis_index("s")
    sbase = c * E_PAD + s * EPT
    dbase = s * EPT

    def issue_idx(chunk, sl):
        sb, db, rows, isem, gsem, ssem = sl
        pltpu.async_copy(src2.at[pl.ds(sbase + chunk * K, K)], sb, isem)
        pltpu.async_copy(dst.at[pl.ds(dbase + chunk * K, K)], db, isem)

    def wait_idx(sl):
        sb, db, rows, isem, gsem, ssem = sl
        pltpu.make_async_copy(src2.at[pl.ds(0, K)], sb, isem).wait()
        pltpu.make_async_copy(dst.at[pl.ds(0, K)], db, isem).wait()

    def issue_gather(sl):
        sb, db, rows, isem, gsem, ssem = sl
        pltpu.async_copy(table.at[sb], rows, gsem)

    def wait_gather(sl):
        sb, db, rows, isem, gsem, ssem = sl
        pltpu.make_async_copy(table.at[sb], rows, gsem).wait()

    def issue_scatter(sl):
        sb, db, rows, isem, gsem, ssem = sl
        pltpu.async_copy(rows, acc.at[db], ssem, add=True)

    def wait_scatter(sl):
        sb, db, rows, isem, gsem, ssem = sl
        pltpu.make_async_copy(rows, acc.at[db], ssem).wait()

    issue_idx(0, slots[0])
    issue_idx(1, slots[1])
    pltpu.sync_copy(z128, acc.at[pl.ds(s * STRIPE, STRIPE)])
    wait_idx(slots[0])
    issue_gather(slots[0])
    plsc.subcore_barrier()

    @pl.loop(0, NCH, step=NSLOT)
    def _(j):
        for b in range(NSLOT):
            jj = j + b
            s0 = slots[b]
            s1 = slots[(b + 1) % NSLOT]
            s2 = slots[(b + 2) % NSLOT]

            @pl.when(jj >= 2)
            def _():
                wait_scatter(s2)

            @pl.when(jj + 2 < NCH)
            def _():
                issue_idx(jj + 2, s2)

            @pl.when(jj + 1 < NCH)
            def _():
                wait_idx(s1)
                issue_gather(s1)

            wait_gather(s0)
            issue_scatter(s0)

    wait_scatter(slots[(NCH - 2) % NSLOT])
    wait_scatter(slots[(NCH - 1) % NSLOT])

    plsc.subcore_barrier()

    sl = pl.ds(s * STRIPE, STRIPE)
    osl = pl.ds(c * N_PAD + s * STRIPE, STRIPE)
    pltpu.sync_copy(acc.at[sl], s_out.at[osl])


def _make_sc_call():
    scratch = [pltpu.VMEM_SHARED((N_PAD, H), jnp.float32)]
    for _ in range(NSLOT):
        scratch.extend([
            pltpu.VMEM((K,), jnp.int32),
            pltpu.VMEM((K,), jnp.int32),
            pltpu.VMEM((K, H), jnp.float32),
            pltpu.SemaphoreType.DMA,
            pltpu.SemaphoreType.DMA,
            pltpu.SemaphoreType.DMA,
        ])
    return pl.kernel(
        _sc_body,
        out_type=jax.ShapeDtypeStruct((NC * N_PAD, H), jnp.float32),
        mesh=_mesh(),
        scratch_types=scratch,
        name="sage_sc_scatter",
    )


EPT_D = E_PAD // (NC * NS)
NCH_D = EPT_D // K


def _sc_deg_body(dst, ones128, z128, deg_out, acc, ones_v, *refs):
    slots = [refs[i * 3:(i + 1) * 3] for i in range(NSLOT)]

    c = lax.axis_index("c")
    s = lax.axis_index("s")
    dbase = (c * NS + s) * EPT_D

    def issue_idx(chunk, sl):
        db, isem, ssem = sl
        pltpu.async_copy(dst.at[pl.ds(dbase + chunk * K, K)], db, isem)

    def wait_idx(sl):
        db, isem, ssem = sl
        pltpu.make_async_copy(dst.at[pl.ds(0, K)], db, isem).wait()

    def issue_scatter(sl):
        db, isem, ssem = sl
        pltpu.async_copy(ones_v, acc.at[db], ssem, add=True)

    def wait_scatter(sl):
        db, isem, ssem = sl
        pltpu.make_async_copy(ones_v, acc.at[db], ssem).wait()

    issue_idx(0, slots[0])
    issue_idx(1, slots[1])
    pltpu.sync_copy(z128, acc.at[pl.ds(s * STRIPE, STRIPE)])
    pltpu.sync_copy(ones128, ones_v)
    plsc.subcore_barrier()

    @pl.loop(0, NCH_D, step=NSLOT)
    def _(j):
        for b in range(NSLOT):
            jj = j + b
            s0 = slots[b]
            s2 = slots[(b + 2) % NSLOT]

            @pl.when(jj >= 2)
            def _():
                wait_scatter(s2)

            @pl.when(jj + 2 < NCH_D)
            def _():
                issue_idx(jj + 2, s2)

            wait_idx(s0)
            issue_scatter(s0)

    wait_scatter(slots[(NCH_D - 2) % NSLOT])
    wait_scatter(slots[(NCH_D - 1) % NSLOT])

    plsc.subcore_barrier()

    sl = pl.ds(s * STRIPE, STRIPE)
    osl = pl.ds(c * N_PAD + s * STRIPE, STRIPE)
    pltpu.sync_copy(acc.at[sl], deg_out.at[osl])


def _make_sc_deg_call():
    scratch = [
        pltpu.VMEM_SHARED((N_PAD, H), jnp.float32),
        pltpu.VMEM((K, H), jnp.float32),
    ]
    for _ in range(NSLOT):
        scratch.extend([
            pltpu.VMEM((K,), jnp.int32),
            pltpu.SemaphoreType.DMA,
            pltpu.SemaphoreType.DMA,
        ])
    return pl.kernel(
        _sc_deg_body,
        out_type=jax.ShapeDtypeStruct((NC * N_PAD, H), jnp.float32),
        mesh=_mesh(),
        scratch_types=scratch,
        name="sage_sc_degree",
    )


def _tc_in_body(x_ref, wn_ref, ws_ref, b_ref, g_ref, p_ref):
    xb = x_ref[...]
    wn = wn_ref[...]
    g_ref[0] = jnp.dot(xb, wn[:, :H], preferred_element_type=jnp.float32)
    g_ref[1] = jnp.dot(xb, wn[:, H:], preferred_element_type=jnp.float32)
    p_ref[...] = (jnp.dot(xb, ws_ref[...], preferred_element_type=jnp.float32)
                  + b_ref[...])


def _tc_in(x, wn, ws, b):
    return pl.pallas_call(
        _tc_in_body,
        grid=(N // BN,),
        in_specs=[
            pl.BlockSpec((BN, D), lambda i: (i, 0)),
            pl.BlockSpec((D, D), lambda i: (0, 0)),
            pl.BlockSpec((D, D), lambda i: (0, 0)),
            pl.BlockSpec((1, D), lambda i: (0, 0)),
        ],
        out_specs=[
            pl.BlockSpec((NC, BN, H), lambda i: (0, i, 0)),
            pl.BlockSpec((BN, D), lambda i: (i, 0)),
        ],
        out_shape=[
            jax.ShapeDtypeStruct((NC, N, H), jnp.float32),
            jax.ShapeDtypeStruct((N, D), jnp.float32),
        ],
        name="sage_tc_in",
    )(x, wn, ws, b)


def _tc_mid_body(p_ref, s_ref, deg_ref, wn_ref, ws_ref, b_ref, g_ref, o_ref):
    deg = deg_ref[0, :, 0:1] + deg_ref[1, :, 0:1]
    rd = 1.0 / jnp.maximum(deg, 1.0)
    sc = jnp.concatenate([s_ref[0], s_ref[1]], axis=1)
    h1 = jnp.maximum(p_ref[...] + sc * rd, 0.0)
    wn = wn_ref[...]
    g_ref[0] = jnp.dot(h1, wn[:, :H], preferred_element_type=jnp.float32)
    g_ref[1] = jnp.dot(h1, wn[:, H:], preferred_element_type=jnp.float32)
    o_ref[...] = (jnp.dot(h1, ws_ref[...], preferred_element_type=jnp.float32)
                  + b_ref[...])


def _tc_mid(p0, s0, deg16, wn, ws, b):
    return pl.pallas_call(
        _tc_mid_body,
        grid=(N // BN,),
        in_specs=[
            pl.BlockSpec((BN, D), lambda i: (i, 0)),
            pl.BlockSpec((NC, BN, H), lambda i: (0, i, 0)),
            pl.BlockSpec((NC, BN, H), lambda i: (0, i, 0)),
            pl.BlockSpec((D, D), lambda i: (0, 0)),
            pl.BlockSpec((D, D), lambda i: (0, 0)),
            pl.BlockSpec((1, D), lambda i: (0, 0)),
        ],
        out_specs=[
            pl.BlockSpec((NC, BN, H), lambda i: (0, i, 0)),
            pl.BlockSpec((BN, D), lambda i: (i, 0)),
        ],
        out_shape=[
            jax.ShapeDtypeStruct((NC, N, H), jnp.float32),
            jax.ShapeDtypeStruct((N, D), jnp.float32),
        ],
        name="sage_tc_mid",
    )(p0, s0, deg16, wn, ws, b)


def _tc_out_body(p_ref, s_ref, deg_ref, o_ref):
    deg = deg_ref[0, :, 0:1] + deg_ref[1, :, 0:1]
    rd = 1.0 / jnp.maximum(deg, 1.0)
    sc = jnp.concatenate([s_ref[0], s_ref[1]], axis=1)
    o_ref[...] = p_ref[...] + sc * rd


def _tc_out(p1, s1, deg16):
    return pl.pallas_call(
        _tc_out_body,
        grid=(N // BN,),
        in_specs=[
            pl.BlockSpec((BN, D), lambda i: (i, 0)),
            pl.BlockSpec((NC, BN, H), lambda i: (0, i, 0)),
            pl.BlockSpec((NC, BN, H), lambda i: (0, i, 0)),
        ],
        out_specs=pl.BlockSpec((BN, D), lambda i: (i, 0)),
        out_shape=jax.ShapeDtypeStruct((N, D), jnp.float32),
        name="sage_tc_out",
    )(p1, s1, deg16)


def kernel(x, edge_index, W_self0, W_neigh0, b0, W_self1, W_neigh1, b1):
    src = edge_index[0]
    dst = edge_index[1]
    pad = E_PAD - E
    src_p = jnp.concatenate([src, jnp.zeros((pad,), jnp.int32)])
    dst_p = jnp.concatenate([dst, jnp.full((pad,), N, jnp.int32)])
    src2 = jnp.concatenate([src_p, src_p + N])

    z128 = jnp.zeros((STRIPE, H), jnp.float32)
    ones128 = jnp.ones((K, H), jnp.float32)

    b0r = b0.reshape(1, D)
    b1r = b1.reshape(1, D)

    sc_scatter = _make_sc_call()
    sc_degree = _make_sc_deg_call()

    deg = sc_degree(dst_p, ones128, z128).reshape(NC, N_PAD, H)
    g0, p0 = _tc_in(x, W_neigh0, W_self0, b0r)
    s0 = sc_scatter(src2, dst_p, g0.reshape(NC * N, H), z128)
    s0 = s0.reshape(NC, N_PAD, H)
    g1, p1 = _tc_mid(p0, s0, deg, W_neigh1, W_self1, b1r)
    s1 = sc_scatter(src2, dst_p, g1.reshape(NC * N, H), z128)
    s1 = s1.reshape(NC, N_PAD, H)
    return _tc_out(p1, s1, deg)
